# baseline (speedup 1.0000x reference)
import functools

import jax
import jax.numpy as jnp
from jax import lax
from jax.experimental import pallas as pl
from jax.experimental.pallas import tpu as pltpu
from jax.experimental.pallas import tpu_sc as plsc

NUM_CORES = 2
NUM_SUBCORES = 16
NW = NUM_CORES * NUM_SUBCORES
LANES = 16
DMA_GROUP = 16


def _sc_gather_pool(xi, catf, emb, ctab_p):
  B = xi.shape[0]
  D = emb.shape[1]
  n_cat = catf.shape[0] // B
  bpw = B // NW
  half = bpw // 2
  assert B % (8 * NW) == 0

  mesh = plsc.VectorSubcoreMesh(core_axis_name="c", subcore_axis_name="s")

  @functools.partial(
      pl.kernel,
      out_type=(
          jax.ShapeDtypeStruct((B, D), jnp.float32),
          jax.ShapeDtypeStruct((B * D,), jnp.float32),
      ),
      mesh=mesh,
      compiler_params=pltpu.CompilerParams(
          use_tc_tiling_on_sc=True, needs_layout_passes=False),
      scratch_types=[
          pltpu.VMEM((bpw,), jnp.int32),
          pltpu.VMEM((half,), jnp.int32),
          pltpu.VMEM((bpw * n_cat,), jnp.int32),
          pltpu.VMEM((half, 128), jnp.float32),
          pltpu.VMEM((bpw * D,), jnp.float32),
          pltpu.SemaphoreType.DMA,
          pltpu.SemaphoreType.DMA,
      ],
  )
  def k(xi_hbm, catf_hbm, emb_hbm, ctab_hbm, uout_hbm, cout_hbm,
        idx_v, cidx_v, catblk_v, tmp_v, acc_v, sem, sem2):
    wid = lax.axis_index("s") * NUM_CORES + lax.axis_index("c")
    base = wid * bpw
    iota = lax.iota(jnp.int32, LANES)
    iota_nc = iota * n_cat

    with jax.named_scope("stage"):
      pltpu.sync_copy(xi_hbm.at[pl.ds(base, bpw)], idx_v)
      pltpu.sync_copy(catf_hbm.at[pl.ds(base * n_cat, bpw * n_cat)], catblk_v)

    with jax.named_scope("ugather"):
      def ugrp(g, _):
        vec = idx_v[pl.ds(g * LANES, LANES)]
        for t in range(LANES):
          pltpu.async_copy(emb_hbm.at[pl.ds(vec[t], 1)],
                           uout_hbm.at[pl.ds(base + g * LANES + t, 1)], sem)
        return 0

      lax.fori_loop(0, bpw // LANES, ugrp, 0)
      pltpu.make_async_copy(
          emb_hbm.at[pl.ds(0, bpw)], uout_hbm.at[pl.ds(base, bpw)],
          sem).wait()

    with jax.named_scope("cat"):
     for h in range(2):
      hbase = h * half
      for j in range(n_cat):

        def ccol(c, _):
          flat = iota_nc + ((hbase + c * LANES) * n_cat + j)
          cidx_v[pl.ds(c * LANES, LANES)] = plsc.load_gather(
              catblk_v, [flat])
          return 0

        lax.fori_loop(0, half // LANES, ccol, 0)
        pltpu.async_copy(ctab_hbm.at[cidx_v], tmp_v, sem2).wait()

        if j == 0:
          def init(i, _):
            for cc in range(D // LANES):
              acc_v[pl.ds((hbase + i) * D + cc * LANES, LANES)] = (
                  tmp_v[i, pl.ds(cc * LANES, LANES)])
            return 0

          lax.fori_loop(0, half, init, 0)
        else:
          def accum(i, _):
            for cc in range(D // LANES):
              v = tmp_v[i, pl.ds(cc * LANES, LANES)]
              plsc.addupdate(
                  acc_v.at[pl.ds((hbase + i) * D + cc * LANES, LANES)], v)
            return 0

          lax.fori_loop(0, half, accum, 0)

    with jax.named_scope("catout"):
      pltpu.sync_copy(acc_v, cout_hbm.at[pl.ds(base * D, bpw * D)])

  return k(xi, catf, emb, ctab_p)


def _tc_mlp(u, cp, numz, w1u, w1c, w1n, b1, w2, b2):
  B, D = u.shape
  H = w1u.shape[1]
  NP = numz.shape[1]
  BLK = 2048
  grid = (B // BLK,)

  def body(u_ref, c_ref, n_ref, w1u_ref, w1c_ref, w1n_ref, b1_ref, w2_ref,
           b2_ref, o_ref):
    h = jnp.dot(u_ref[...], w1u_ref[...], preferred_element_type=jnp.float32)
    h = h + jnp.dot(c_ref[...], w1c_ref[...],
                    preferred_element_type=jnp.float32)
    h = h + jnp.dot(n_ref[...], w1n_ref[...],
                    preferred_element_type=jnp.float32)
    h = jnp.maximum(h + b1_ref[...], 0.0)
    o_ref[...] = (jnp.dot(h, w2_ref[...], preferred_element_type=jnp.float32)
                  + b2_ref[0, 0])

  return pl.pallas_call(
      body,
      grid=grid,
      in_specs=[
          pl.BlockSpec((BLK, D), lambda i: (i, 0)),
          pl.BlockSpec((BLK, D), lambda i: (i, 0)),
          pl.BlockSpec((BLK, NP), lambda i: (i, 0)),
          pl.BlockSpec((D, H), lambda i: (0, 0)),
          pl.BlockSpec((D, H), lambda i: (0, 0)),
          pl.BlockSpec((NP, H), lambda i: (0, 0)),
          pl.BlockSpec((1, H), lambda i: (0, 0)),
          pl.BlockSpec((H, 1), lambda i: (0, 0)),
          pl.BlockSpec(memory_space=pltpu.SMEM),
      ],
      out_specs=pl.BlockSpec((BLK, 1), lambda i: (i, 0)),
      out_shape=jax.ShapeDtypeStruct((B, 1), jnp.float32),
  )(u, cp, numz, w1u, w1c, w1n, b1, w2, b2)


def kernel(x, category, numeric, emb, cat_table, W1, b1, W2, b2):
  B = x.shape[0]
  D = emb.shape[1]
  n_num = numeric.shape[1]

  xi = x[:, 0].astype(jnp.int32)
  catf = category.astype(jnp.int32).reshape(-1)
  ctab_p = jnp.pad(cat_table, ((0, 0), (0, 128 - D)))

  user_emb, cflat = _sc_gather_pool(xi, catf, emb, ctab_p)
  cat_pooled = cflat.reshape(B, D)

  np_pad = 16
  numz = jnp.pad(numeric, ((0, 0), (0, np_pad - n_num)))
  w1u = W1[:D]
  w1c = W1[D:2 * D]
  w1n = jnp.pad(W1[2 * D:], ((0, np_pad - n_num), (0, 0)))
  b1r = b1.reshape(1, -1)
  b2r = b2.reshape(1, 1)

  return _tc_mlp(user_emb, cat_pooled, numz, w1u, w1c, w1n, b1r, W2, b2r)


# speedup vs baseline: 2.1826x; 2.1826x over previous
import functools

import jax
import jax.numpy as jnp
from jax import lax
from jax.experimental import pallas as pl
from jax.experimental.pallas import tpu as pltpu
from jax.experimental.pallas import tpu_sc as plsc

NUM_CORES = 2
NUM_SUBCORES = 16
NW = NUM_CORES * NUM_SUBCORES
LANES = 16


def _sc_cat_pool(catf, cat_table, B):
  D = cat_table.shape[1]
  n_cat = catf.shape[0] // B
  bpw = B // NW

  mesh = plsc.VectorSubcoreMesh(core_axis_name="c", subcore_axis_name="s")

  @functools.partial(
      pl.kernel,
      out_type=jax.ShapeDtypeStruct((B, D), jnp.float32),
      mesh=mesh,
      compiler_params=pltpu.CompilerParams(
          use_tc_tiling_on_sc=False, needs_layout_passes=False),
      scratch_types=[
          pltpu.VMEM((bpw,), jnp.int32),
          pltpu.VMEM((bpw * n_cat,), jnp.int32),
          pltpu.VMEM((bpw, D), jnp.float32),
          pltpu.VMEM((bpw, D), jnp.float32),
          pltpu.SemaphoreType.DMA,
      ],
  )
  def k(catf_hbm, ctab_hbm, cout_hbm, cidx_v, catblk_v, tmp_v, acc_v, sem):
    wid = lax.axis_index("s") * NUM_CORES + lax.axis_index("c")
    base = wid * bpw
    iota = lax.iota(jnp.int32, LANES)
    iota_nc = iota * n_cat

    with jax.named_scope("cstage"):
      pltpu.sync_copy(catf_hbm.at[pl.ds(base * n_cat, bpw * n_cat)], catblk_v)

    with jax.named_scope("cat"):
      for j in range(n_cat):

        def ccol(c, _):
          flat = iota_nc + ((c * LANES) * n_cat + j)
          cidx_v[pl.ds(c * LANES, LANES)] = plsc.load_gather(
              catblk_v, [flat])
          return 0

        lax.fori_loop(0, bpw // LANES, ccol, 0)
        if j == 0:
          pltpu.async_copy(ctab_hbm.at[cidx_v], acc_v, sem).wait()
        else:
          pltpu.async_copy(ctab_hbm.at[cidx_v], tmp_v, sem).wait()

          def accum(i, _):
            for cc in range(D // LANES):
              v = tmp_v[i, pl.ds(cc * LANES, LANES)]
              plsc.addupdate(acc_v.at[i, pl.ds(cc * LANES, LANES)], v)
            return 0

          lax.fori_loop(0, bpw, accum, 0)

    with jax.named_scope("cout"):
      pltpu.sync_copy(acc_v, cout_hbm.at[pl.ds(base, bpw)])

  return k(catf, cat_table)


def _sc_user_gather(xi, emb):
  B = xi.shape[0]
  D = emb.shape[1]
  bpw = B // NW

  mesh = plsc.VectorSubcoreMesh(core_axis_name="c", subcore_axis_name="s")

  @functools.partial(
      pl.kernel,
      out_type=jax.ShapeDtypeStruct((B, D), jnp.float32),
      mesh=mesh,
      compiler_params=pltpu.CompilerParams(
          use_tc_tiling_on_sc=True, needs_layout_passes=False),
      scratch_types=[
          pltpu.VMEM((bpw,), jnp.int32),
          pltpu.VMEM((bpw, D), jnp.float32),
          pltpu.SemaphoreType.DMA,
      ],
  )
  def k(xi_hbm, emb_hbm, uout_hbm, idx_v, urows_v, sem):
    wid = lax.axis_index("s") * NUM_CORES + lax.axis_index("c")
    base = wid * bpw

    with jax.named_scope("ustage"):
      pltpu.sync_copy(xi_hbm.at[pl.ds(base, bpw)], idx_v)

    with jax.named_scope("ugather"):
      def ugrp(g, _):
        vec = idx_v[pl.ds(g * LANES, LANES)]
        for t in range(LANES):
          pltpu.async_copy(emb_hbm.at[pl.ds(vec[t], 1)],
                           urows_v.at[pl.ds(g * LANES + t, 1)], sem)
        return 0

      lax.fori_loop(0, bpw // LANES, ugrp, 0)
      pltpu.make_async_copy(
          emb_hbm.at[pl.ds(0, bpw)], urows_v, sem).wait()

    with jax.named_scope("uout"):
      pltpu.sync_copy(urows_v, uout_hbm.at[pl.ds(base, bpw)])

  return k(xi, emb)


def _tc_mlp(u, cp, numz, w1u, w1c, w1n, b1, w2, b2):
  B, D = u.shape
  H = w1u.shape[1]
  NP = numz.shape[1]
  BLK = 2048
  grid = (B // BLK,)

  def body(u_ref, c_ref, n_ref, w1u_ref, w1c_ref, w1n_ref, b1_ref, w2_ref,
           b2_ref, o_ref):
    h = jnp.dot(u_ref[...], w1u_ref[...], preferred_element_type=jnp.float32)
    h = h + jnp.dot(c_ref[...], w1c_ref[...],
                    preferred_element_type=jnp.float32)
    h = h + jnp.dot(n_ref[...], w1n_ref[...],
                    preferred_element_type=jnp.float32)
    h = jnp.maximum(h + b1_ref[...], 0.0)
    o_ref[...] = (jnp.dot(h, w2_ref[...], preferred_element_type=jnp.float32)
                  + b2_ref[0, 0])

  return pl.pallas_call(
      body,
      grid=grid,
      in_specs=[
          pl.BlockSpec((BLK, D), lambda i: (i, 0)),
          pl.BlockSpec((BLK, D), lambda i: (i, 0)),
          pl.BlockSpec((BLK, NP), lambda i: (i, 0)),
          pl.BlockSpec((D, H), lambda i: (0, 0)),
          pl.BlockSpec((D, H), lambda i: (0, 0)),
          pl.BlockSpec((NP, H), lambda i: (0, 0)),
          pl.BlockSpec((1, H), lambda i: (0, 0)),
          pl.BlockSpec((H, 1), lambda i: (0, 0)),
          pl.BlockSpec(memory_space=pltpu.SMEM),
      ],
      out_specs=pl.BlockSpec((BLK, 1), lambda i: (i, 0)),
      out_shape=jax.ShapeDtypeStruct((B, 1), jnp.float32),
  )(u, cp, numz, w1u, w1c, w1n, b1, w2, b2)


def kernel(x, category, numeric, emb, cat_table, W1, b1, W2, b2):
  B = x.shape[0]
  D = emb.shape[1]
  n_num = numeric.shape[1]

  xi = x[:, 0].astype(jnp.int32)
  catf = category.astype(jnp.int32).reshape(-1)

  cat_pooled = _sc_cat_pool(catf, cat_table, B)
  user_emb = _sc_user_gather(xi, emb)

  np_pad = 16
  numz = jnp.pad(numeric, ((0, 0), (0, np_pad - n_num)))
  w1u = W1[:D]
  w1c = W1[D:2 * D]
  w1n = jnp.pad(W1[2 * D:], ((0, np_pad - n_num), (0, 0)))
  b1r = b1.reshape(1, -1)
  b2r = b2.reshape(1, 1)

  return _tc_mlp(user_emb, cat_pooled, numz, w1u, w1c, w1n, b1r, W2, b2r)


# speedup vs baseline: 2.4540x; 1.1244x over previous
import functools

import jax
import jax.numpy as jnp
from jax import lax
from jax.experimental import pallas as pl
from jax.experimental.pallas import tpu as pltpu
from jax.experimental.pallas import tpu_sc as plsc

NUM_CORES = 2
NUM_SUBCORES = 16
NW = NUM_CORES * NUM_SUBCORES
LANES = 16


def _sc_cat_pool(catf, cat_table, B):
  D = cat_table.shape[1]
  n_cat = catf.shape[0] // B
  bpw = B // NW

  mesh = plsc.VectorSubcoreMesh(core_axis_name="c", subcore_axis_name="s")

  @functools.partial(
      pl.kernel,
      out_type=jax.ShapeDtypeStruct((B, D), jnp.float32),
      mesh=mesh,
      compiler_params=pltpu.CompilerParams(
          use_tc_tiling_on_sc=False, needs_layout_passes=False),
      scratch_types=[
          pltpu.VMEM((bpw,), jnp.int32),
          pltpu.VMEM((bpw * n_cat,), jnp.int32),
          pltpu.VMEM((bpw, D), jnp.float32),
          pltpu.VMEM((bpw, D), jnp.float32),
          pltpu.SemaphoreType.DMA,
      ],
  )
  def k(catf_hbm, ctab_hbm, cout_hbm, cidx_v, catblk_v, tmp_v, acc_v, sem):
    wid = lax.axis_index("s") * NUM_CORES + lax.axis_index("c")
    base = wid * bpw
    iota = lax.iota(jnp.int32, LANES)
    iota_nc = iota * n_cat

    with jax.named_scope("cstage"):
      pltpu.sync_copy(catf_hbm.at[pl.ds(base * n_cat, bpw * n_cat)], catblk_v)

    with jax.named_scope("cat"):
      for j in range(n_cat):

        def ccol(c, _):
          flat = iota_nc + ((c * LANES) * n_cat + j)
          cidx_v[pl.ds(c * LANES, LANES)] = plsc.load_gather(
              catblk_v, [flat])
          return 0

        lax.fori_loop(0, bpw // LANES, ccol, 0)
        if j == 0:
          pltpu.async_copy(ctab_hbm.at[cidx_v], acc_v, sem).wait()
        else:
          pltpu.async_copy(ctab_hbm.at[cidx_v], tmp_v, sem).wait()

          def accum(i, _):
            for cc in range(D // LANES):
              v = tmp_v[i, pl.ds(cc * LANES, LANES)]
              plsc.addupdate(acc_v.at[i, pl.ds(cc * LANES, LANES)], v)
            return 0

          lax.fori_loop(0, bpw, accum, 0)

    with jax.named_scope("cout"):
      pltpu.sync_copy(acc_v, cout_hbm.at[pl.ds(base, bpw)])

  return k(catf, cat_table)


def _sc_user_gather(xi, emb):
  B = xi.shape[0]
  D = emb.shape[1]
  bpw = B // NW

  mesh = plsc.VectorSubcoreMesh(core_axis_name="c", subcore_axis_name="s")

  @functools.partial(
      pl.kernel,
      out_type=jax.ShapeDtypeStruct((B, D), jnp.float32),
      mesh=mesh,
      compiler_params=pltpu.CompilerParams(
          use_tc_tiling_on_sc=True, needs_layout_passes=False),
      scratch_types=[
          pltpu.VMEM((bpw,), jnp.int32),
          pltpu.VMEM((bpw, D), jnp.float32),
          pltpu.SemaphoreType.DMA,
      ],
  )
  def k(xi_hbm, emb_hbm, uout_hbm, idx_v, urows_v, sem):
    wid = lax.axis_index("s") * NUM_CORES + lax.axis_index("c")
    base = wid * bpw

    with jax.named_scope("ustage"):
      pltpu.sync_copy(xi_hbm.at[pl.ds(base, bpw)], idx_v)

    with jax.named_scope("ugather"):
      def ugrp(g, _):
        vec = idx_v[pl.ds(g * LANES, LANES)]
        for t in range(LANES):
          pltpu.async_copy(emb_hbm.at[pl.ds(vec[t], 1)],
                           urows_v.at[pl.ds(g * LANES + t, 1)], sem)
        return 0

      lax.fori_loop(0, bpw // LANES, ugrp, 0)
      pltpu.make_async_copy(
          emb_hbm.at[pl.ds(0, bpw)], urows_v, sem).wait()

    with jax.named_scope("uout"):
      pltpu.sync_copy(urows_v, uout_hbm.at[pl.ds(base, bpw)])

  return k(xi, emb)


def _sc_user_sweep(xi, embt, tailf):
  B = xi.shape[0]
  D = embt.shape[0]
  V = embt.shape[1]
  CW = 512
  RPW = 31232
  NCH = RPW // CW
  TAIL0 = NW * RPW
  TAIL1 = (V // 64) * 64 - CW

  mesh = plsc.VectorSubcoreMesh(core_axis_name="c", subcore_axis_name="s")

  @functools.partial(
      pl.kernel,
      out_type=jax.ShapeDtypeStruct((B, D), jnp.float32),
      mesh=mesh,
      compiler_params=pltpu.CompilerParams(
          use_tc_tiling_on_sc=True, needs_layout_passes=False),
      scratch_types=[
          pltpu.VMEM((2048,), jnp.int32),
          pltpu.VMEM((B + 16,), jnp.int32),
          pltpu.VMEM((B + 16,), jnp.int32),
          pltpu.VMEM((B + 16,), jnp.int32),
          pltpu.VMEM((B + 16,), jnp.int32),
          pltpu.VMEM((D, CW), jnp.float32),
          pltpu.VMEM((D * 64,), jnp.float32),
          pltpu.VMEM((16, D), jnp.float32),
          pltpu.SemaphoreType.DMA,
          pltpu.SemaphoreType.DMA,
      ],
  )
  def k(xi_hbm, embt_hbm, tailf_hbm, uout_hbm, xip_v, ml_v, pl_v, sml_v,
        spl_v, chunk_v, tail_v, rows_v, sem, osem):
    wid = lax.axis_index("s") * NUM_CORES + lax.axis_index("c")
    iota = lax.iota(jnp.int32, LANES)
    lo = wid * RPW
    hi = jnp.where(wid == NW - 1, V, lo + RPW)

    with jax.named_scope("scan"):
      def piece(pc, cnt):
        pltpu.sync_copy(xi_hbm.at[pl.ds(pc * 2048, 2048)], xip_v)

        def grp(g, cnt):
          r = xip_v[pl.ds(g * LANES, LANES)]
          m = (r >= lo) & (r < hi)
          plsc.store_compressed(ml_v.at[pl.ds(cnt, LANES)], r, mask=m)
          plsc.store_compressed(
              pl_v.at[pl.ds(cnt, LANES)], iota + (pc * 2048 + g * LANES),
              mask=m)
          return cnt + plsc.all_reduce_population_count(m)[0]

        return lax.fori_loop(0, 2048 // LANES, grp, cnt)

      cnt = lax.fori_loop(0, B // 2048, piece, jnp.int32(0))

    def process_chunk(cb, cw, flat_tail=False):
      def rgrp(g, sc):
        r = ml_v[pl.ds(g * LANES, LANES)]
        p = pl_v[pl.ds(g * LANES, LANES)]
        m = (r >= cb) & (r < cb + cw) & ((iota + g * LANES) < cnt)
        plsc.store_compressed(sml_v.at[pl.ds(sc, LANES)], r, mask=m)
        plsc.store_compressed(spl_v.at[pl.ds(sc, LANES)], p, mask=m)
        return sc + plsc.all_reduce_population_count(m)[0]

      ngr = (cnt + LANES - 1) // LANES
      sc = lax.fori_loop(0, ngr, rgrp, jnp.int32(0))

      def egrp(g, _):
        rvec = sml_v[pl.ds(g * LANES, LANES)] - cb
        pvec = spl_v[pl.ds(g * LANES, LANES)]
        for t in range(LANES):
          @pl.when(g * LANES + t < sc)
          def _():
            cl = jnp.full((LANES,), rvec[t], jnp.int32)
            for kk in range(D // LANES):
              if flat_tail:
                rows_v[t, pl.ds(kk * LANES, LANES)] = plsc.load_gather(
                    tail_v, [(iota + kk * LANES) * 64 + cl])
              else:
                rows_v[t, pl.ds(kk * LANES, LANES)] = plsc.load_gather(
                    chunk_v, [iota + kk * LANES, cl])
            pltpu.async_copy(rows_v.at[pl.ds(t, 1)],
                             uout_hbm.at[pl.ds(pvec[t], 1)], osem)
        rem = sc - g * LANES
        nn = jnp.minimum(rem, LANES)

        def drain(_, x):
          pltpu.make_async_copy(
              rows_v.at[pl.ds(0, 1)], uout_hbm.at[pl.ds(0, 1)], osem).wait()
          return x

        lax.fori_loop(0, nn, drain, 0)
        return 0

      negr = (sc + LANES - 1) // LANES
      lax.fori_loop(0, negr, egrp, 0)

    with jax.named_scope("sweep"):
      def chunk_iter(ci, _):
        cb = lo + ci * CW
        pltpu.sync_copy(embt_hbm.at[:, pl.ds(cb, CW)], chunk_v)
        process_chunk(cb, CW)
        return 0

      lax.fori_loop(0, NCH, chunk_iter, 0)

      @pl.when(wid == NW - 1)
      def _():
        pltpu.sync_copy(embt_hbm.at[:, pl.ds(TAIL0, CW)], chunk_v)
        process_chunk(jnp.int32(TAIL0), CW)
        pltpu.sync_copy(tailf_hbm, tail_v)
        process_chunk(jnp.int32(TAIL0 + CW), 64, flat_tail=True)

  return k(xi, embt, tailf)


def _tc_mlp(u, cp, numz, w1u, w1c, w1n, b1, w2, b2):
  B, D = u.shape
  H = w1u.shape[1]
  NP = numz.shape[1]
  BLK = 2048
  grid = (B // BLK,)

  def body(u_ref, c_ref, n_ref, w1u_ref, w1c_ref, w1n_ref, b1_ref, w2_ref,
           b2_ref, o_ref):
    h = jnp.dot(u_ref[...], w1u_ref[...], preferred_element_type=jnp.float32)
    h = h + jnp.dot(c_ref[...], w1c_ref[...],
                    preferred_element_type=jnp.float32)
    h = h + jnp.dot(n_ref[...], w1n_ref[...],
                    preferred_element_type=jnp.float32)
    h = jnp.maximum(h + b1_ref[...], 0.0)
    o_ref[...] = (jnp.dot(h, w2_ref[...], preferred_element_type=jnp.float32)
                  + b2_ref[0, 0])

  return pl.pallas_call(
      body,
      grid=grid,
      in_specs=[
          pl.BlockSpec((BLK, D), lambda i: (i, 0)),
          pl.BlockSpec((BLK, D), lambda i: (i, 0)),
          pl.BlockSpec((BLK, NP), lambda i: (i, 0)),
          pl.BlockSpec((D, H), lambda i: (0, 0)),
          pl.BlockSpec((D, H), lambda i: (0, 0)),
          pl.BlockSpec((NP, H), lambda i: (0, 0)),
          pl.BlockSpec((1, H), lambda i: (0, 0)),
          pl.BlockSpec((H, 1), lambda i: (0, 0)),
          pl.BlockSpec(memory_space=pltpu.SMEM),
      ],
      out_specs=pl.BlockSpec((BLK, 1), lambda i: (i, 0)),
      out_shape=jax.ShapeDtypeStruct((B, 1), jnp.float32),
  )(u, cp, numz, w1u, w1c, w1n, b1, w2, b2)


def kernel(x, category, numeric, emb, cat_table, W1, b1, W2, b2):
  B = x.shape[0]
  D = emb.shape[1]
  n_num = numeric.shape[1]

  xi = x[:, 0].astype(jnp.int32)
  catf = category.astype(jnp.int32).reshape(-1)

  cat_pooled = _sc_cat_pool(catf, cat_table, B)
  tailf = emb[999936:].T.reshape(-1)
  user_emb = _sc_user_sweep(xi, emb.T, tailf)

  np_pad = 16
  numz = jnp.pad(numeric, ((0, 0), (0, np_pad - n_num)))
  w1u = W1[:D]
  w1c = W1[D:2 * D]
  w1n = jnp.pad(W1[2 * D:], ((0, np_pad - n_num), (0, 0)))
  b1r = b1.reshape(1, -1)
  b2r = b2.reshape(1, 1)

  return _tc_mlp(user_emb, cat_pooled, numz, w1u, w1c, w1n, b1r, W2, b2r)


# speedup vs baseline: 2.7683x; 1.1281x over previous
import functools

import jax
import jax.numpy as jnp
from jax import lax
from jax.experimental import pallas as pl
from jax.experimental.pallas import tpu as pltpu
from jax.experimental.pallas import tpu_sc as plsc

NUM_CORES = 2
NUM_SUBCORES = 16
NW = NUM_CORES * NUM_SUBCORES
LANES = 16


def _sc_cat_pool(catf, cat_table, B):
  D = cat_table.shape[1]
  n_cat = catf.shape[0] // B
  bpw = B // NW

  mesh = plsc.VectorSubcoreMesh(core_axis_name="c", subcore_axis_name="s")

  @functools.partial(
      pl.kernel,
      out_type=jax.ShapeDtypeStruct((B, D), jnp.float32),
      mesh=mesh,
      compiler_params=pltpu.CompilerParams(
          use_tc_tiling_on_sc=False, needs_layout_passes=False),
      scratch_types=[
          pltpu.VMEM((2, bpw), jnp.int32),
          pltpu.VMEM((bpw * n_cat,), jnp.int32),
          pltpu.VMEM((2, bpw, D), jnp.float32),
          pltpu.VMEM((bpw, D), jnp.float32),
          pltpu.SemaphoreType.DMA,
      ],
  )
  def k(catf_hbm, ctab_hbm, cout_hbm, cidx_v, catblk_v, tmp_v, acc_v, sem):
    wid = lax.axis_index("s") * NUM_CORES + lax.axis_index("c")
    base = wid * bpw
    iota = lax.iota(jnp.int32, LANES)
    iota_nc = iota * n_cat

    with jax.named_scope("cstage"):
      pltpu.sync_copy(catf_hbm.at[pl.ds(base * n_cat, bpw * n_cat)], catblk_v)

    def build_idx(j, b):
      def ccol(c, _):
        flat = iota_nc + ((c * LANES) * n_cat + j)
        cidx_v[b, pl.ds(c * LANES, LANES)] = plsc.load_gather(
            catblk_v, [flat])
        return 0

      lax.fori_loop(0, bpw // LANES, ccol, 0)

    def consume(j, b):
      pltpu.make_async_copy(
          ctab_hbm.at[cidx_v.at[0]], tmp_v.at[0], sem).wait()
      if j == 0:
        def init(i, _):
          for cc in range(D // LANES):
            acc_v[i, pl.ds(cc * LANES, LANES)] = (
                tmp_v[b, i, pl.ds(cc * LANES, LANES)])
          return 0

        lax.fori_loop(0, bpw, init, 0)
      else:
        def accum(i, _):
          for cc in range(D // LANES):
            v = tmp_v[b, i, pl.ds(cc * LANES, LANES)]
            plsc.addupdate(acc_v.at[i, pl.ds(cc * LANES, LANES)], v)
          return 0

        lax.fori_loop(0, bpw, accum, 0)

    with jax.named_scope("cat"):
      build_idx(0, 0)
      pltpu.async_copy(ctab_hbm.at[cidx_v.at[0]], tmp_v.at[0], sem)
      for j in range(1, n_cat):
        b = j % 2
        build_idx(j, b)
        pltpu.async_copy(ctab_hbm.at[cidx_v.at[b]], tmp_v.at[b], sem)
        consume(j - 1, (j - 1) % 2)
      consume(n_cat - 1, (n_cat - 1) % 2)

    with jax.named_scope("cout"):
      pltpu.sync_copy(acc_v, cout_hbm.at[pl.ds(base, bpw)])

  return k(catf, cat_table)


def _sc_user_gather(xi, emb):
  B = xi.shape[0]
  D = emb.shape[1]
  bpw = B // NW

  mesh = plsc.VectorSubcoreMesh(core_axis_name="c", subcore_axis_name="s")

  @functools.partial(
      pl.kernel,
      out_type=jax.ShapeDtypeStruct((B, D), jnp.float32),
      mesh=mesh,
      compiler_params=pltpu.CompilerParams(
          use_tc_tiling_on_sc=True, needs_layout_passes=False),
      scratch_types=[
          pltpu.VMEM((bpw,), jnp.int32),
          pltpu.VMEM((bpw, D), jnp.float32),
          pltpu.SemaphoreType.DMA,
      ],
  )
  def k(xi_hbm, emb_hbm, uout_hbm, idx_v, urows_v, sem):
    wid = lax.axis_index("s") * NUM_CORES + lax.axis_index("c")
    base = wid * bpw

    with jax.named_scope("ustage"):
      pltpu.sync_copy(xi_hbm.at[pl.ds(base, bpw)], idx_v)

    with jax.named_scope("ugather"):
      def ugrp(g, _):
        vec = idx_v[pl.ds(g * LANES, LANES)]
        for t in range(LANES):
          pltpu.async_copy(emb_hbm.at[pl.ds(vec[t], 1)],
                           urows_v.at[pl.ds(g * LANES + t, 1)], sem)
        return 0

      lax.fori_loop(0, bpw // LANES, ugrp, 0)
      pltpu.make_async_copy(
          emb_hbm.at[pl.ds(0, bpw)], urows_v, sem).wait()

    with jax.named_scope("uout"):
      pltpu.sync_copy(urows_v, uout_hbm.at[pl.ds(base, bpw)])

  return k(xi, emb)


def _sc_user_sweep(xi, embt, tailf):
  B = xi.shape[0]
  D = embt.shape[0]
  V = embt.shape[1]
  CW = 512
  RPW = 31232
  NCH = RPW // CW
  TAIL0 = NW * RPW
  TAIL1 = (V // 64) * 64 - CW

  mesh = plsc.VectorSubcoreMesh(core_axis_name="c", subcore_axis_name="s")

  @functools.partial(
      pl.kernel,
      out_type=jax.ShapeDtypeStruct((B, D), jnp.float32),
      mesh=mesh,
      compiler_params=pltpu.CompilerParams(
          use_tc_tiling_on_sc=True, needs_layout_passes=False),
      scratch_types=[
          pltpu.VMEM((2048,), jnp.int32),
          pltpu.VMEM((B + 16,), jnp.int32),
          pltpu.VMEM((B + 16,), jnp.int32),
          pltpu.VMEM((B + 16,), jnp.int32),
          pltpu.VMEM((B + 16,), jnp.int32),
          pltpu.VMEM((D, CW), jnp.float32),
          pltpu.VMEM((D * 64,), jnp.float32),
          pltpu.VMEM((16, D), jnp.float32),
          pltpu.SemaphoreType.DMA,
          pltpu.SemaphoreType.DMA,
      ],
  )
  def k(xi_hbm, embt_hbm, tailf_hbm, uout_hbm, xip_v, ml_v, pl_v, sml_v,
        spl_v, chunk_v, tail_v, rows_v, sem, osem):
    wid = lax.axis_index("s") * NUM_CORES + lax.axis_index("c")
    iota = lax.iota(jnp.int32, LANES)
    lo = wid * RPW
    hi = jnp.where(wid == NW - 1, V, lo + RPW)

    with jax.named_scope("scan"):
      def piece(pc, cnt):
        pltpu.sync_copy(xi_hbm.at[pl.ds(pc * 2048, 2048)], xip_v)

        def grp(g, cnt):
          r = xip_v[pl.ds(g * LANES, LANES)]
          m = (r >= lo) & (r < hi)
          plsc.store_compressed(ml_v.at[pl.ds(cnt, LANES)], r, mask=m)
          plsc.store_compressed(
              pl_v.at[pl.ds(cnt, LANES)], iota + (pc * 2048 + g * LANES),
              mask=m)
          return cnt + plsc.all_reduce_population_count(m)[0]

        return lax.fori_loop(0, 2048 // LANES, grp, cnt)

      cnt = lax.fori_loop(0, B // 2048, piece, jnp.int32(0))

    def process_chunk(cb, cw, flat_tail=False):
      def rgrp(g, sc):
        r = ml_v[pl.ds(g * LANES, LANES)]
        p = pl_v[pl.ds(g * LANES, LANES)]
        m = (r >= cb) & (r < cb + cw) & ((iota + g * LANES) < cnt)
        plsc.store_compressed(sml_v.at[pl.ds(sc, LANES)], r, mask=m)
        plsc.store_compressed(spl_v.at[pl.ds(sc, LANES)], p, mask=m)
        return sc + plsc.all_reduce_population_count(m)[0]

      ngr = (cnt + LANES - 1) // LANES
      sc = lax.fori_loop(0, ngr, rgrp, jnp.int32(0))

      def egrp(g, _):
        rvec = sml_v[pl.ds(g * LANES, LANES)] - cb
        pvec = spl_v[pl.ds(g * LANES, LANES)]
        for t in range(LANES):
          @pl.when(g * LANES + t < sc)
          def _():
            cl = jnp.full((LANES,), rvec[t], jnp.int32)
            for kk in range(D // LANES):
              if flat_tail:
                rows_v[t, pl.ds(kk * LANES, LANES)] = plsc.load_gather(
                    tail_v, [(iota + kk * LANES) * 64 + cl])
              else:
                rows_v[t, pl.ds(kk * LANES, LANES)] = plsc.load_gather(
                    chunk_v, [iota + kk * LANES, cl])
            pltpu.async_copy(rows_v.at[pl.ds(t, 1)],
                             uout_hbm.at[pl.ds(pvec[t], 1)], osem)
        rem = sc - g * LANES
        nn = jnp.minimum(rem, LANES)

        def drain(_, x):
          pltpu.make_async_copy(
              rows_v.at[pl.ds(0, 1)], uout_hbm.at[pl.ds(0, 1)], osem).wait()
          return x

        lax.fori_loop(0, nn, drain, 0)
        return 0

      negr = (sc + LANES - 1) // LANES
      lax.fori_loop(0, negr, egrp, 0)

    with jax.named_scope("sweep"):
      def chunk_iter(ci, _):
        cb = lo + ci * CW
        pltpu.sync_copy(embt_hbm.at[:, pl.ds(cb, CW)], chunk_v)
        process_chunk(cb, CW)
        return 0

      lax.fori_loop(0, NCH, chunk_iter, 0)

      @pl.when(wid == NW - 1)
      def _():
        pltpu.sync_copy(embt_hbm.at[:, pl.ds(TAIL0, CW)], chunk_v)
        process_chunk(jnp.int32(TAIL0), CW)
        pltpu.sync_copy(tailf_hbm, tail_v)
        process_chunk(jnp.int32(TAIL0 + CW), 64, flat_tail=True)

  return k(xi, embt, tailf)


def _tc_mlp(u, cp, numz, w1u, w1c, w1n, b1, w2, b2):
  B, D = u.shape
  H = w1u.shape[1]
  NP = numz.shape[1]
  BLK = 2048
  grid = (B // BLK,)

  def body(u_ref, c_ref, n_ref, w1u_ref, w1c_ref, w1n_ref, b1_ref, w2_ref,
           b2_ref, o_ref):
    h = jnp.dot(u_ref[...], w1u_ref[...], preferred_element_type=jnp.float32)
    h = h + jnp.dot(c_ref[...], w1c_ref[...],
                    preferred_element_type=jnp.float32)
    h = h + jnp.dot(n_ref[...], w1n_ref[...],
                    preferred_element_type=jnp.float32)
    h = jnp.maximum(h + b1_ref[...], 0.0)
    o_ref[...] = (jnp.dot(h, w2_ref[...], preferred_element_type=jnp.float32)
                  + b2_ref[0, 0])

  return pl.pallas_call(
      body,
      grid=grid,
      in_specs=[
          pl.BlockSpec((BLK, D), lambda i: (i, 0)),
          pl.BlockSpec((BLK, D), lambda i: (i, 0)),
          pl.BlockSpec((BLK, NP), lambda i: (i, 0)),
          pl.BlockSpec((D, H), lambda i: (0, 0)),
          pl.BlockSpec((D, H), lambda i: (0, 0)),
          pl.BlockSpec((NP, H), lambda i: (0, 0)),
          pl.BlockSpec((1, H), lambda i: (0, 0)),
          pl.BlockSpec((H, 1), lambda i: (0, 0)),
          pl.BlockSpec(memory_space=pltpu.SMEM),
      ],
      out_specs=pl.BlockSpec((BLK, 1), lambda i: (i, 0)),
      out_shape=jax.ShapeDtypeStruct((B, 1), jnp.float32),
  )(u, cp, numz, w1u, w1c, w1n, b1, w2, b2)


def kernel(x, category, numeric, emb, cat_table, W1, b1, W2, b2):
  B = x.shape[0]
  D = emb.shape[1]
  n_num = numeric.shape[1]

  xi = x[:, 0].astype(jnp.int32)
  catf = category.astype(jnp.int32).reshape(-1)

  cat_pooled = _sc_cat_pool(catf, cat_table, B)
  tailf = emb[999936:].T.reshape(-1)
  user_emb = _sc_user_sweep(xi, emb.T, tailf)

  np_pad = 16
  numz = jnp.pad(numeric, ((0, 0), (0, np_pad - n_num)))
  w1u = W1[:D]
  w1c = W1[D:2 * D]
  w1n = jnp.pad(W1[2 * D:], ((0, np_pad - n_num), (0, 0)))
  b1r = b1.reshape(1, -1)
  b2r = b2.reshape(1, 1)

  return _tc_mlp(user_emb, cat_pooled, numz, w1u, w1c, w1n, b1r, W2, b2r)


# speedup vs baseline: 3.1266x; 1.1294x over previous
import functools

import jax
import jax.numpy as jnp
from jax import lax
from jax.experimental import pallas as pl
from jax.experimental.pallas import tpu as pltpu
from jax.experimental.pallas import tpu_sc as plsc

NUM_CORES = 2
NUM_SUBCORES = 16
NW = NUM_CORES * NUM_SUBCORES
LANES = 16


def _sc_cat_pool(catf, cat_table, B):
  D = cat_table.shape[1]
  n_cat = catf.shape[0] // B
  bpw = B // NW

  mesh = plsc.VectorSubcoreMesh(core_axis_name="c", subcore_axis_name="s")

  @functools.partial(
      pl.kernel,
      out_type=jax.ShapeDtypeStruct((B, D), jnp.float32),
      mesh=mesh,
      compiler_params=pltpu.CompilerParams(
          use_tc_tiling_on_sc=False, needs_layout_passes=False),
      scratch_types=[
          pltpu.VMEM((2, bpw), jnp.int32),
          pltpu.VMEM((bpw * n_cat,), jnp.int32),
          pltpu.VMEM((2, bpw, D), jnp.float32),
          pltpu.VMEM((bpw, D), jnp.float32),
          pltpu.SemaphoreType.DMA,
      ],
  )
  def k(catf_hbm, ctab_hbm, cout_hbm, cidx_v, catblk_v, tmp_v, acc_v, sem):
    wid = lax.axis_index("s") * NUM_CORES + lax.axis_index("c")
    base = wid * bpw
    iota = lax.iota(jnp.int32, LANES)
    iota_nc = iota * n_cat

    with jax.named_scope("cstage"):
      pltpu.sync_copy(catf_hbm.at[pl.ds(base * n_cat, bpw * n_cat)], catblk_v)

    def build_idx(j, b):
      def ccol(c, _):
        flat = iota_nc + ((c * LANES) * n_cat + j)
        cidx_v[b, pl.ds(c * LANES, LANES)] = plsc.load_gather(
            catblk_v, [flat])
        return 0

      lax.fori_loop(0, bpw // LANES, ccol, 0)

    def consume(j, b):
      pltpu.make_async_copy(
          ctab_hbm.at[cidx_v.at[0]], tmp_v.at[0], sem).wait()
      if j == 0:
        def init(i, _):
          for cc in range(D // LANES):
            acc_v[i, pl.ds(cc * LANES, LANES)] = (
                tmp_v[b, i, pl.ds(cc * LANES, LANES)])
          return 0

        lax.fori_loop(0, bpw, init, 0)
      else:
        def accum(i, _):
          for cc in range(D // LANES):
            v = tmp_v[b, i, pl.ds(cc * LANES, LANES)]
            plsc.addupdate(acc_v.at[i, pl.ds(cc * LANES, LANES)], v)
          return 0

        lax.fori_loop(0, bpw, accum, 0)

    with jax.named_scope("cat"):
      build_idx(0, 0)
      pltpu.async_copy(ctab_hbm.at[cidx_v.at[0]], tmp_v.at[0], sem)
      for j in range(1, n_cat):
        b = j % 2
        build_idx(j, b)
        pltpu.async_copy(ctab_hbm.at[cidx_v.at[b]], tmp_v.at[b], sem)
        consume(j - 1, (j - 1) % 2)
      consume(n_cat - 1, (n_cat - 1) % 2)

    with jax.named_scope("cout"):
      pltpu.sync_copy(acc_v, cout_hbm.at[pl.ds(base, bpw)])

  return k(catf, cat_table)


def _sc_user_gather(xi, emb):
  B = xi.shape[0]
  D = emb.shape[1]
  bpw = B // NW

  mesh = plsc.VectorSubcoreMesh(core_axis_name="c", subcore_axis_name="s")

  @functools.partial(
      pl.kernel,
      out_type=jax.ShapeDtypeStruct((B, D), jnp.float32),
      mesh=mesh,
      compiler_params=pltpu.CompilerParams(
          use_tc_tiling_on_sc=True, needs_layout_passes=False),
      scratch_types=[
          pltpu.VMEM((bpw,), jnp.int32),
          pltpu.VMEM((bpw, D), jnp.float32),
          pltpu.SemaphoreType.DMA,
      ],
  )
  def k(xi_hbm, emb_hbm, uout_hbm, idx_v, urows_v, sem):
    wid = lax.axis_index("s") * NUM_CORES + lax.axis_index("c")
    base = wid * bpw

    with jax.named_scope("ustage"):
      pltpu.sync_copy(xi_hbm.at[pl.ds(base, bpw)], idx_v)

    with jax.named_scope("ugather"):
      def ugrp(g, _):
        vec = idx_v[pl.ds(g * LANES, LANES)]
        for t in range(LANES):
          pltpu.async_copy(emb_hbm.at[pl.ds(vec[t], 1)],
                           urows_v.at[pl.ds(g * LANES + t, 1)], sem)
        return 0

      lax.fori_loop(0, bpw // LANES, ugrp, 0)
      pltpu.make_async_copy(
          emb_hbm.at[pl.ds(0, bpw)], urows_v, sem).wait()

    with jax.named_scope("uout"):
      pltpu.sync_copy(urows_v, uout_hbm.at[pl.ds(base, bpw)])

  return k(xi, emb)


def _sc_user_sweep(xi, embt, tailf):
  B = xi.shape[0]
  D = embt.shape[0]
  V = embt.shape[1]
  CW = 256
  RPW = 31232
  NCH = RPW // CW
  TAIL0 = NW * RPW
  TAIL1 = (V // 64) * 64 - CW

  mesh = plsc.VectorSubcoreMesh(core_axis_name="c", subcore_axis_name="s")

  @functools.partial(
      pl.kernel,
      out_type=jax.ShapeDtypeStruct((B, D), jnp.float32),
      mesh=mesh,
      compiler_params=pltpu.CompilerParams(
          use_tc_tiling_on_sc=True, needs_layout_passes=False),
      scratch_types=[
          pltpu.VMEM((2048,), jnp.int32),
          pltpu.VMEM((B + 16,), jnp.int32),
          pltpu.VMEM((B + 16,), jnp.int32),
          pltpu.VMEM((B + 16,), jnp.int32),
          pltpu.VMEM((B + 16,), jnp.int32),
          pltpu.VMEM((2, D, CW), jnp.float32),
          pltpu.VMEM((D * 64,), jnp.float32),
          pltpu.VMEM((16, D), jnp.float32),
          pltpu.SemaphoreType.DMA,
          pltpu.SemaphoreType.DMA,
      ],
  )
  def k(xi_hbm, embt_hbm, tailf_hbm, uout_hbm, xip_v, ml_v, pl_v, sml_v,
        spl_v, chunk_v, tail_v, rows_v, sem, osem):
    wid = lax.axis_index("s") * NUM_CORES + lax.axis_index("c")
    iota = lax.iota(jnp.int32, LANES)
    lo = wid * RPW
    hi = jnp.where(wid == NW - 1, V, lo + RPW)

    with jax.named_scope("scan"):
      def piece(pc, cnt):
        pltpu.sync_copy(xi_hbm.at[pl.ds(pc * 2048, 2048)], xip_v)

        def grp(g, cnt):
          r = xip_v[pl.ds(g * LANES, LANES)]
          m = (r >= lo) & (r < hi)
          plsc.store_compressed(ml_v.at[pl.ds(cnt, LANES)], r, mask=m)
          plsc.store_compressed(
              pl_v.at[pl.ds(cnt, LANES)], iota + (pc * 2048 + g * LANES),
              mask=m)
          return cnt + plsc.all_reduce_population_count(m)[0]

        return lax.fori_loop(0, 2048 // LANES, grp, cnt)

      cnt = lax.fori_loop(0, B // 2048, piece, jnp.int32(0))

    def process_chunk(cref, cb, cw, flat_tail=False):
      def rgrp(g, sc):
        r = ml_v[pl.ds(g * LANES, LANES)]
        p = pl_v[pl.ds(g * LANES, LANES)]
        m = (r >= cb) & (r < cb + cw) & ((iota + g * LANES) < cnt)
        plsc.store_compressed(sml_v.at[pl.ds(sc, LANES)], r, mask=m)
        plsc.store_compressed(spl_v.at[pl.ds(sc, LANES)], p, mask=m)
        return sc + plsc.all_reduce_population_count(m)[0]

      ngr = (cnt + LANES - 1) // LANES
      sc = lax.fori_loop(0, ngr, rgrp, jnp.int32(0))

      def egrp(g, _):
        rvec = sml_v[pl.ds(g * LANES, LANES)] - cb
        pvec = spl_v[pl.ds(g * LANES, LANES)]
        for t in range(LANES):
          @pl.when(g * LANES + t < sc)
          def _():
            cl = jnp.full((LANES,), rvec[t], jnp.int32)
            for kk in range(D // LANES):
              if flat_tail:
                rows_v[t, pl.ds(kk * LANES, LANES)] = plsc.load_gather(
                    tail_v, [(iota + kk * LANES) * 64 + cl])
              else:
                rows_v[t, pl.ds(kk * LANES, LANES)] = plsc.load_gather(
                    cref, [iota + kk * LANES, cl])
            pltpu.async_copy(rows_v.at[pl.ds(t, 1)],
                             uout_hbm.at[pl.ds(pvec[t], 1)], osem)
        rem = sc - g * LANES
        nn = jnp.minimum(rem, LANES)

        def drain(_, x):
          pltpu.make_async_copy(
              rows_v.at[pl.ds(0, 1)], uout_hbm.at[pl.ds(0, 1)], osem).wait()
          return x

        lax.fori_loop(0, nn, drain, 0)
        return 0

      negr = (sc + LANES - 1) // LANES
      lax.fori_loop(0, negr, egrp, 0)

    with jax.named_scope("sweep"):
      pltpu.async_copy(embt_hbm.at[:, pl.ds(lo, CW)], chunk_v.at[0], sem)

      def chunk_pair(cp, _):
        for b in range(2):
          ci = cp * 2 + b

          @pl.when(ci + 1 < NCH)
          def _():
            pltpu.async_copy(embt_hbm.at[:, pl.ds(lo + (ci + 1) * CW, CW)],
                             chunk_v.at[1 - b], sem)

          pltpu.make_async_copy(
              embt_hbm.at[:, pl.ds(lo, CW)], chunk_v.at[0], sem).wait()
          process_chunk(chunk_v.at[b], lo + ci * CW, CW)
        return 0

      lax.fori_loop(0, NCH // 2, chunk_pair, 0)

      @pl.when(wid == NW - 1)
      def _():
        pltpu.sync_copy(embt_hbm.at[:, pl.ds(TAIL0, CW)], chunk_v.at[0])
        process_chunk(chunk_v.at[0], jnp.int32(TAIL0), CW)
        pltpu.sync_copy(embt_hbm.at[:, pl.ds(TAIL0 + CW, CW)], chunk_v.at[1])
        process_chunk(chunk_v.at[1], jnp.int32(TAIL0 + CW), CW)
        pltpu.sync_copy(tailf_hbm, tail_v)
        process_chunk(chunk_v.at[0], jnp.int32(TAIL0 + 2 * CW), 64,
                      flat_tail=True)

  return k(xi, embt, tailf)


def _tc_mlp(u, cp, numz, w1u, w1c, w1n, b1, w2, b2):
  B, D = u.shape
  H = w1u.shape[1]
  NP = numz.shape[1]
  BLK = 2048
  grid = (B // BLK,)

  def body(u_ref, c_ref, n_ref, w1u_ref, w1c_ref, w1n_ref, b1_ref, w2_ref,
           b2_ref, o_ref):
    h = jnp.dot(u_ref[...], w1u_ref[...], preferred_element_type=jnp.float32)
    h = h + jnp.dot(c_ref[...], w1c_ref[...],
                    preferred_element_type=jnp.float32)
    h = h + jnp.dot(n_ref[...], w1n_ref[...],
                    preferred_element_type=jnp.float32)
    h = jnp.maximum(h + b1_ref[...], 0.0)
    o_ref[...] = (jnp.dot(h, w2_ref[...], preferred_element_type=jnp.float32)
                  + b2_ref[0, 0])

  return pl.pallas_call(
      body,
      grid=grid,
      in_specs=[
          pl.BlockSpec((BLK, D), lambda i: (i, 0)),
          pl.BlockSpec((BLK, D), lambda i: (i, 0)),
          pl.BlockSpec((BLK, NP), lambda i: (i, 0)),
          pl.BlockSpec((D, H), lambda i: (0, 0)),
          pl.BlockSpec((D, H), lambda i: (0, 0)),
          pl.BlockSpec((NP, H), lambda i: (0, 0)),
          pl.BlockSpec((1, H), lambda i: (0, 0)),
          pl.BlockSpec((H, 1), lambda i: (0, 0)),
          pl.BlockSpec(memory_space=pltpu.SMEM),
      ],
      out_specs=pl.BlockSpec((BLK, 1), lambda i: (i, 0)),
      out_shape=jax.ShapeDtypeStruct((B, 1), jnp.float32),
  )(u, cp, numz, w1u, w1c, w1n, b1, w2, b2)


def kernel(x, category, numeric, emb, cat_table, W1, b1, W2, b2):
  B = x.shape[0]
  D = emb.shape[1]
  n_num = numeric.shape[1]

  xi = x[:, 0].astype(jnp.int32)
  catf = category.astype(jnp.int32).reshape(-1)

  cat_pooled = _sc_cat_pool(catf, cat_table, B)
  tailf = emb[999936:].T.reshape(-1)
  user_emb = _sc_user_sweep(xi, emb.T, tailf)

  np_pad = 16
  numz = jnp.pad(numeric, ((0, 0), (0, np_pad - n_num)))
  w1u = W1[:D]
  w1c = W1[D:2 * D]
  w1n = jnp.pad(W1[2 * D:], ((0, np_pad - n_num), (0, 0)))
  b1r = b1.reshape(1, -1)
  b2r = b2.reshape(1, 1)

  return _tc_mlp(user_emb, cat_pooled, numz, w1u, w1c, w1n, b1r, W2, b2r)
